# baseline (speedup 1.0000x reference)
import functools

import jax
import jax.numpy as jnp
from jax import lax
from jax.experimental import pallas as pl
from jax.experimental.pallas import tpu as pltpu
from jax.experimental.pallas import tpu_sc as plsc

SEQ = 200
BATCH = 4096
EMBED = 128
NC = 2
NS = 16
NW = NC * NS
BPW = BATCH // NW
LANES = 16
NCH = EMBED // LANES
NBUF = 4
UNROLL = 8
INV_SEQ = 1.0 / SEQ


def _w2v_body(sent, table, out, idx_v, b0, b1, b2, b3, tacc, s0, s1, s2, s3):
    c = lax.axis_index("c")
    s = lax.axis_index("s")
    wid = s * NC + c
    base = wid * BPW
    bufs = (b0, b1, b2, b3)
    sems = (s0, s1, s2, s3)

    pltpu.sync_copy(sent.at[:, pl.ds(base, BPW)], idx_v)

    def gather(l, k):
        pltpu.async_copy(table.at[idx_v.at[l]], bufs[k], sems[k])

    def wait_gather(l, k):
        pltpu.make_async_copy(table.at[idx_v.at[l]], bufs[k], sems[k]).wait()

    def tec_accumulate(buf, init):
        def abody(r8, carry):
            for ur in range(UNROLL):
                r = UNROLL * r8 + ur
                for ch in range(NCH):
                    sl = pl.ds(ch * LANES, LANES)
                    if init:
                        tacc[r, sl] = buf[r, sl]
                    else:
                        plsc.addupdate(tacc.at[r, sl], buf[r, sl])
            return carry

        lax.fori_loop(0, BPW // UNROLL, abody, 0)

    for k in range(NBUF):
        gather(k, k)

    wait_gather(0, 0)
    tec_accumulate(bufs[0], init=True)
    gather(NBUF, 0)

    for k in range(1, NBUF):
        wait_gather(k, k)
        tec_accumulate(bufs[k], init=False)
        gather(k + NBUF, k)

    def gbody(g, carry):
        for k in range(NBUF):
            l = NBUF * g + NBUF + k
            wait_gather(l, k)
            tec_accumulate(bufs[k], init=False)
            gather(l + NBUF, k)
        return carry

    lax.fori_loop(0, (SEQ - 2 * NBUF) // NBUF, gbody, 0)

    for k in range(NBUF):
        l = SEQ - NBUF + k
        wait_gather(l, k)
        tec_accumulate(bufs[k], init=False)

    def sbody(r, carry):
        for ch in range(NCH):
            sl = pl.ds(ch * LANES, LANES)
            tacc[r, sl] = tacc[r, sl] * INV_SEQ
        return carry

    lax.fori_loop(0, BPW, sbody, 0)
    pltpu.sync_copy(tacc, out.at[pl.ds(base, BPW)])


@jax.jit
def kernel(sentence, table):
    sentence = sentence.astype(jnp.int32)
    mesh = plsc.VectorSubcoreMesh(
        core_axis_name="c", subcore_axis_name="s", num_cores=NC, num_subcores=NS
    )
    k = functools.partial(
        pl.kernel,
        out_type=jax.ShapeDtypeStruct((BATCH, EMBED), jnp.float32),
        mesh=mesh,
        scratch_types=[
            pltpu.VMEM((SEQ, BPW), jnp.int32),
            pltpu.VMEM((BPW, EMBED), jnp.float32),
            pltpu.VMEM((BPW, EMBED), jnp.float32),
            pltpu.VMEM((BPW, EMBED), jnp.float32),
            pltpu.VMEM((BPW, EMBED), jnp.float32),
            pltpu.VMEM((BPW, EMBED), jnp.float32),
            pltpu.SemaphoreType.DMA,
            pltpu.SemaphoreType.DMA,
            pltpu.SemaphoreType.DMA,
            pltpu.SemaphoreType.DMA,
        ],
    )(_w2v_body)
    return k(sentence, table)


# speedup vs baseline: 1.1461x; 1.1461x over previous
import functools

import jax
import jax.numpy as jnp
from jax import lax
from jax.experimental import pallas as pl
from jax.experimental.pallas import tpu as pltpu
from jax.experimental.pallas import tpu_sc as plsc

SEQ = 200
BATCH = 4096
EMBED = 128
NC = 2
NS = 16
NW = NC * NS
BPW = BATCH // NW
LANES = 16
NCH = EMBED // LANES
NBUF = 4
UNROLL = 8
INV_SEQ = 1.0 / SEQ


def _w2v_body(sent, table, out, idx_v, b0, b1, b2, b3, tacc, s0, s1, s2, s3):
    c = lax.axis_index("c")
    s = lax.axis_index("s")
    wid = s * NC + c
    base = wid * BPW
    bufs = (b0, b1, b2, b3)
    sems = (s0, s1, s2, s3)

    pltpu.sync_copy(sent.at[:, pl.ds(base, BPW)], idx_v)

    def gather(l, k):
        pltpu.async_copy(table.at[idx_v.at[l]], bufs[k], sems[k])

    def wait_gather(l, k):
        pltpu.make_async_copy(table.at[idx_v.at[l]], bufs[k], sems[k]).wait()

    def tec_accumulate_pair(ba, bb, init):
        def abody(r4, carry):
            for ur in range(UNROLL):
                r = UNROLL * r4 + ur
                for ch in range(NCH):
                    sl = pl.ds(ch * LANES, LANES)
                    v = ba[r, sl] + bb[r, sl]
                    if init:
                        tacc[r, sl] = v
                    else:
                        plsc.addupdate(tacc.at[r, sl], v)
            return carry

        lax.fori_loop(0, BPW // UNROLL, abody, 0)

    def process_pair(l, k, init=False):
        wait_gather(l, k)
        wait_gather(l + 1, k + 1)
        tec_accumulate_pair(bufs[k], bufs[k + 1], init)

    def refill_pair(l, k):
        gather(l, k)
        gather(l + 1, k + 1)

    for k in range(NBUF):
        gather(k, k)

    process_pair(0, 0, init=True)
    refill_pair(NBUF, 0)
    process_pair(2, 2)
    refill_pair(NBUF + 2, 2)

    def gbody(g, carry):
        l = NBUF * g + NBUF
        process_pair(l, 0)
        refill_pair(l + NBUF, 0)
        process_pair(l + 2, 2)
        refill_pair(l + NBUF + 2, 2)
        return carry

    lax.fori_loop(0, (SEQ - 2 * NBUF) // NBUF, gbody, 0)

    process_pair(SEQ - NBUF, 0)
    process_pair(SEQ - NBUF + 2, 2)

    def sbody(r, carry):
        for ch in range(NCH):
            sl = pl.ds(ch * LANES, LANES)
            tacc[r, sl] = tacc[r, sl] * INV_SEQ
        return carry

    lax.fori_loop(0, BPW, sbody, 0)
    pltpu.sync_copy(tacc, out.at[pl.ds(base, BPW)])


@jax.jit
def kernel(sentence, table):
    sentence = sentence.astype(jnp.int32)
    mesh = plsc.VectorSubcoreMesh(
        core_axis_name="c", subcore_axis_name="s", num_cores=NC, num_subcores=NS
    )
    k = functools.partial(
        pl.kernel,
        out_type=jax.ShapeDtypeStruct((BATCH, EMBED), jnp.float32),
        mesh=mesh,
        scratch_types=[
            pltpu.VMEM((SEQ, BPW), jnp.int32),
            pltpu.VMEM((BPW, EMBED), jnp.float32),
            pltpu.VMEM((BPW, EMBED), jnp.float32),
            pltpu.VMEM((BPW, EMBED), jnp.float32),
            pltpu.VMEM((BPW, EMBED), jnp.float32),
            pltpu.VMEM((BPW, EMBED), jnp.float32),
            pltpu.SemaphoreType.DMA,
            pltpu.SemaphoreType.DMA,
            pltpu.SemaphoreType.DMA,
            pltpu.SemaphoreType.DMA,
        ],
    )(_w2v_body)
    return k(sentence, table)


# speedup vs baseline: 1.2348x; 1.0774x over previous
import functools

import jax
import jax.numpy as jnp
from jax import lax
from jax.experimental import pallas as pl
from jax.experimental.pallas import tpu as pltpu
from jax.experimental.pallas import tpu_sc as plsc

SEQ = 200
BATCH = 4096
EMBED = 128
NC = 2
NS = 16
NW = NC * NS
BPW = BATCH // NW
HPW = BPW // 2
LANES = 16
NCH = EMBED // LANES
NQ = SEQ // 4
UNROLL = 4
INV_SEQ = 1.0 / SEQ


def _w2v_body(sent, table, out, idx_v, b0, b1, b2, b3, b4, b5, b6, b7, tacc,
              s0, s1, s2, s3, s4, s5, s6, s7):
    c = lax.axis_index("c")
    s = lax.axis_index("s")
    wid = s * NC + c
    base = wid * BPW
    bufs = (b0, b1, b2, b3, b4, b5, b6, b7)
    sems = (s0, s1, s2, s3, s4, s5, s6, s7)

    pltpu.sync_copy(sent.at[:, pl.ds(base, BPW)], idx_v)

    def gather_quad(q, h):
        for i in range(4):
            pltpu.async_copy(
                table.at[idx_v.at[4 * q + i, pl.ds(h * HPW, HPW)]],
                bufs[4 * h + i], sems[4 * h + i])

    def wait_quad(q, h):
        for i in range(4):
            pltpu.make_async_copy(
                table.at[idx_v.at[4 * q + i, pl.ds(h * HPW, HPW)]],
                bufs[4 * h + i], sems[4 * h + i]).wait()

    def accumulate_quad(h, init):
        ba, bb, bc, bd = bufs[4 * h], bufs[4 * h + 1], bufs[4 * h + 2], \
            bufs[4 * h + 3]
        hr = h * HPW

        def abody(r4, carry):
            for ur in range(UNROLL):
                r = UNROLL * r4 + ur
                for ch in range(NCH):
                    sl = pl.ds(ch * LANES, LANES)
                    v = (ba[r, sl] + bb[r, sl]) + (bc[r, sl] + bd[r, sl])
                    if init:
                        tacc[hr + r, sl] = v
                    else:
                        plsc.addupdate(tacc.at[hr + r, sl], v)
            return carry

        lax.fori_loop(0, HPW // UNROLL, abody, 0)

    gather_quad(0, 0)
    gather_quad(0, 1)

    wait_quad(0, 0)
    accumulate_quad(0, init=True)
    gather_quad(1, 0)
    wait_quad(0, 1)
    accumulate_quad(1, init=True)
    gather_quad(1, 1)

    def gbody(g, carry):
        q = g + 1
        wait_quad(q, 0)
        accumulate_quad(0, init=False)
        gather_quad(q + 1, 0)
        wait_quad(q, 1)
        accumulate_quad(1, init=False)
        gather_quad(q + 1, 1)
        return carry

    lax.fori_loop(0, NQ - 2, gbody, 0)

    wait_quad(NQ - 1, 0)
    accumulate_quad(0, init=False)
    wait_quad(NQ - 1, 1)
    accumulate_quad(1, init=False)

    def sbody(r, carry):
        for ch in range(NCH):
            sl = pl.ds(ch * LANES, LANES)
            tacc[r, sl] = tacc[r, sl] * INV_SEQ
        return carry

    lax.fori_loop(0, BPW, sbody, 0)
    pltpu.sync_copy(tacc, out.at[pl.ds(base, BPW)])


@jax.jit
def kernel(sentence, table):
    sentence = sentence.astype(jnp.int32)
    mesh = plsc.VectorSubcoreMesh(
        core_axis_name="c", subcore_axis_name="s", num_cores=NC, num_subcores=NS
    )
    k = functools.partial(
        pl.kernel,
        out_type=jax.ShapeDtypeStruct((BATCH, EMBED), jnp.float32),
        mesh=mesh,
        scratch_types=(
            [pltpu.VMEM((SEQ, BPW), jnp.int32)]
            + [pltpu.VMEM((HPW, EMBED), jnp.float32)] * 8
            + [pltpu.VMEM((BPW, EMBED), jnp.float32)]
            + [pltpu.SemaphoreType.DMA] * 8
        ),
    )(_w2v_body)
    return k(sentence, table)
